# baseline (speedup 1.0000x reference)
import functools

import jax
import jax.numpy as jnp
from jax import lax
from jax.experimental import pallas as pl
from jax.experimental.pallas import tpu as pltpu
from jax.experimental.pallas import tpu_sc as plsc

N = 10000
E = 320000
D = 128

NC = 2
NS = 16
NW = NC * NS

B = 128
S = 80
CH = 5
S_C = S // CH
EPW = S * B
E_PAD = NW * EPW
TOTAL_S = E_PAD // B
PER_S = TOTAL_S // NS

K0 = 40
K1 = PER_S - K0
SB_C = 8

RPT = 640
N_PAD = NS * RPT


def _sc_scatter_body(h_hbm, src_hbm, dst_hbm, zrows_hbm, out_hbm,
                     srcv, dstv, buf_a, buf_b, sem_a, sem_b, agg_sh):
    c = lax.axis_index("c")
    s = lax.axis_index("s")
    start = s * PER_S + c * K0
    n_chunks = (K0 + c * (K1 - K0)) // SB_C

    pltpu.sync_copy(zrows_hbm, agg_sh.at[pl.ds(s * RPT, RPT)])

    plsc.subcore_barrier()

    def chunk(ci, carry):
        base = start + ci * SB_C
        pltpu.sync_copy(src_hbm.at[pl.ds(base, SB_C)], srcv)
        pltpu.sync_copy(dst_hbm.at[pl.ds(base, SB_C)], dstv)
        pltpu.async_copy(h_hbm.at[srcv.at[0]], buf_a, sem_a)

        def step(k, c2):
            j = 2 * k
            pltpu.make_async_copy(h_hbm.at[srcv.at[j]], buf_a, sem_a).wait()
            pltpu.async_copy(h_hbm.at[srcv.at[j + 1]], buf_b, sem_b)
            pltpu.sync_copy(buf_a, agg_sh.at[dstv.at[j]], add=True)
            pltpu.make_async_copy(h_hbm.at[srcv.at[j + 1]], buf_b, sem_b).wait()
            j2 = jnp.minimum(j + 2, SB_C - 1)
            pltpu.async_copy(h_hbm.at[srcv.at[j2]], buf_a, sem_a)
            pltpu.sync_copy(buf_b, agg_sh.at[dstv.at[j + 1]], add=True)
            return c2

        lax.fori_loop(0, SB_C // 2, step, 0)
        pltpu.make_async_copy(h_hbm.at[srcv.at[SB_C - 1]], buf_a, sem_a).wait()
        return carry

    lax.fori_loop(0, n_chunks, chunk, 0)

    plsc.subcore_barrier()

    rows = pl.ds(s * RPT, RPT)
    pltpu.sync_copy(agg_sh.at[rows], out_hbm.at[c, rows])


def _make_sc_scatter():
    mesh = plsc.VectorSubcoreMesh(core_axis_name="c", subcore_axis_name="s")
    return pl.kernel(
        _sc_scatter_body,
        out_type=[jax.ShapeDtypeStruct((NC, N_PAD, D), jnp.float32)],
        mesh=mesh,
        scratch_types=[
            pltpu.VMEM((SB_C, B), jnp.int32),
            pltpu.VMEM((SB_C, B), jnp.int32),
            pltpu.VMEM((B, D), jnp.float32),
            pltpu.VMEM((B, D), jnp.float32),
            pltpu.SemaphoreType.DMA,
            pltpu.SemaphoreType.DMA,
            pltpu.VMEM_SHARED((N_PAD, D), jnp.float32),
        ],
    )


def _sc_degree_body(dst_hbm, zrows_hbm, ones_hbm, deg_hbm,
                    dstv, ones_v, deg_sh):
    c = lax.axis_index("c")
    s = lax.axis_index("s")
    w = s * NC + c

    pltpu.sync_copy(dst_hbm.at[w], dstv)
    pltpu.sync_copy(zrows_hbm, deg_sh.at[pl.ds(s * RPT, RPT)])
    pltpu.sync_copy(ones_hbm, ones_v)

    plsc.subcore_barrier()

    def step(j, carry):
        pltpu.sync_copy(ones_v, deg_sh.at[dstv.at[j]], add=True)
        return carry

    lax.fori_loop(0, S, step, 0)

    plsc.subcore_barrier()

    rows = pl.ds(s * RPT, RPT)
    pltpu.sync_copy(deg_sh.at[rows], deg_hbm.at[c, rows])


def _make_sc_degree():
    mesh = plsc.VectorSubcoreMesh(core_axis_name="c", subcore_axis_name="s")
    return pl.kernel(
        _sc_degree_body,
        out_type=[jax.ShapeDtypeStruct((NC, N_PAD, D), jnp.float32)],
        mesh=mesh,
        scratch_types=[
            pltpu.VMEM((S, B), jnp.int32),
            pltpu.VMEM((B, D), jnp.float32),
            pltpu.VMEM_SHARED((N_PAD, D), jnp.float32),
        ],
    )


def _leaky(h):
    return jnp.where(h >= 0, h, 0.1 * h)


def _tc_dense_body(with_act, p_ref, d_ref, h_ref, Wl_ref, Wr_ref, b_ref,
                   LW_ref, lb_ref, o_ref):
    deg = d_ref[0, :, 0:1] + d_ref[1, :, 0:1]
    agg = (p_ref[0] + p_ref[1]) / jnp.maximum(deg, 1.0)
    t = (jnp.dot(agg, Wl_ref[...], preferred_element_type=jnp.float32)
         + b_ref[...]
         + jnp.dot(h_ref[...], Wr_ref[...], preferred_element_type=jnp.float32))
    if with_act:
        t = _leaky(t)
    t = jnp.dot(t, LW_ref[...], preferred_element_type=jnp.float32) + lb_ref[...]
    if with_act:
        t = _leaky(t)
    o_ref[...] = t


def _make_tc_dense(with_act, BN=1000):
    grid = (N // BN,)
    return pl.pallas_call(
        functools.partial(_tc_dense_body, with_act),
        grid=grid,
        in_specs=[
            pl.BlockSpec((NC, BN, D), lambda i: (0, i, 0)),
            pl.BlockSpec((NC, BN, D), lambda i: (0, i, 0)),
            pl.BlockSpec((BN, D), lambda i: (i, 0)),
            pl.BlockSpec((D, D), lambda i: (0, 0)),
            pl.BlockSpec((D, D), lambda i: (0, 0)),
            pl.BlockSpec((1, D), lambda i: (0, 0)),
            pl.BlockSpec((D, D), lambda i: (0, 0)),
            pl.BlockSpec((1, D), lambda i: (0, 0)),
        ],
        out_specs=pl.BlockSpec((BN, D), lambda i: (i, 0)),
        out_shape=jax.ShapeDtypeStruct((N, D), jnp.float32),
    )


_sc_scatter = _make_sc_scatter()
_sc_degree = _make_sc_degree()
_tc_dense_act = _make_tc_dense(True)
_tc_dense_noact = _make_tc_dense(False)


def kernel(x, edge_index,
           conv_Wl0, conv_Wr0, conv_b0, lin_W0, lin_b0,
           conv_Wl1, conv_Wr1, conv_b1, lin_W1, lin_b1,
           conv_Wl2, conv_Wr2, conv_b2, lin_W2, lin_b2):
    src = edge_index[0]
    dst = edge_index[1]
    npad = E_PAD - E
    src_pad = jnp.concatenate([src, jnp.zeros((npad,), jnp.int32)])
    dst_pad = jnp.concatenate([dst, jnp.full((npad,), N_PAD - 1, jnp.int32)])
    src_steps = src_pad.reshape(TOTAL_S, B)
    dst_steps = dst_pad.reshape(TOTAL_S, B)
    dst_slabs = dst_pad.reshape(NW, S, B)

    zrows = jnp.zeros((RPT, D), jnp.float32)
    ones = jnp.ones((B, D), jnp.float32)

    b0 = conv_b0.reshape(1, D)
    b1 = conv_b1.reshape(1, D)
    b2 = conv_b2.reshape(1, D)
    lb0 = lin_b0.reshape(1, D)
    lb1 = lin_b1.reshape(1, D)
    lb2 = lin_b2.reshape(1, D)

    (degp,) = _sc_degree(dst_slabs, zrows, ones)
    (p0,) = _sc_scatter(x, src_steps, dst_steps, zrows)
    h1 = _tc_dense_act(p0, degp, x, conv_Wl0, conv_Wr0, b0, lin_W0, lb0)
    (p1,) = _sc_scatter(h1, src_steps, dst_steps, zrows)
    h2 = _tc_dense_act(p1, degp, h1, conv_Wl1, conv_Wr1, b1, lin_W1, lb1)
    (p2,) = _sc_scatter(h2, src_steps, dst_steps, zrows)
    out = _tc_dense_noact(p2, degp, h2, conv_Wl2, conv_Wr2, b2, lin_W2, lb2)
    return out


# speedup vs baseline: 1.3013x; 1.3013x over previous
import functools

import jax
import jax.numpy as jnp
from jax import lax
from jax.experimental import pallas as pl
from jax.experimental.pallas import tpu as pltpu
from jax.experimental.pallas import tpu_sc as plsc

N = 10000
E = 320000
D = 128

NC = 2
NS = 16
NW = NC * NS

B = 128
S = 80
CH = 5
S_C = S // CH
EPW = S * B
E_PAD = NW * EPW
TOTAL_S = E_PAD // B
PER_S = TOTAL_S // NS

K0 = 120
K1 = PER_S - K0
SB_C = 8

RPT = 640
N_PAD = NS * RPT


def _sc_scatter_body(h_hbm, src_hbm, dst_hbm, zrows_hbm, out_hbm,
                     srcv, dstv, buf_a, buf_b, sem_a, sem_b, agg_sh):
    c = lax.axis_index("c")
    s = lax.axis_index("s")
    start = s * PER_S + c * K0
    n_chunks = (K0 + c * (K1 - K0)) // SB_C

    pltpu.sync_copy(zrows_hbm, agg_sh.at[pl.ds(s * RPT, RPT)])

    plsc.subcore_barrier()

    def chunk(ci, carry):
        base = start + ci * SB_C
        pltpu.sync_copy(src_hbm.at[pl.ds(base, SB_C)], srcv)
        pltpu.sync_copy(dst_hbm.at[pl.ds(base, SB_C)], dstv)
        pltpu.async_copy(h_hbm.at[srcv.at[0]], buf_a, sem_a)

        def step(k, c2):
            j = 2 * k
            pltpu.make_async_copy(h_hbm.at[srcv.at[j]], buf_a, sem_a).wait()
            pltpu.async_copy(h_hbm.at[srcv.at[j + 1]], buf_b, sem_b)
            pltpu.sync_copy(buf_a, agg_sh.at[dstv.at[j]], add=True)
            pltpu.make_async_copy(h_hbm.at[srcv.at[j + 1]], buf_b, sem_b).wait()
            j2 = jnp.minimum(j + 2, SB_C - 1)
            pltpu.async_copy(h_hbm.at[srcv.at[j2]], buf_a, sem_a)
            pltpu.sync_copy(buf_b, agg_sh.at[dstv.at[j + 1]], add=True)
            return c2

        lax.fori_loop(0, SB_C // 2, step, 0)
        pltpu.make_async_copy(h_hbm.at[srcv.at[SB_C - 1]], buf_a, sem_a).wait()
        return carry

    lax.fori_loop(0, n_chunks, chunk, 0)

    plsc.subcore_barrier()

    rows = pl.ds(s * RPT, RPT)
    pltpu.sync_copy(agg_sh.at[rows], out_hbm.at[c, rows])


def _make_sc_scatter():
    mesh = plsc.VectorSubcoreMesh(core_axis_name="c", subcore_axis_name="s")
    return pl.kernel(
        _sc_scatter_body,
        out_type=[jax.ShapeDtypeStruct((NC, N_PAD, D), jnp.float32)],
        mesh=mesh,
        scratch_types=[
            pltpu.VMEM((SB_C, B), jnp.int32),
            pltpu.VMEM((SB_C, B), jnp.int32),
            pltpu.VMEM((B, D), jnp.float32),
            pltpu.VMEM((B, D), jnp.float32),
            pltpu.SemaphoreType.DMA,
            pltpu.SemaphoreType.DMA,
            pltpu.VMEM_SHARED((N_PAD, D), jnp.float32),
        ],
    )


def _sc_degree_body(dst_hbm, zrows_hbm, ones_hbm, deg_hbm,
                    dstv, ones_v, deg_sh):
    c = lax.axis_index("c")
    s = lax.axis_index("s")
    w = s * NC + c

    pltpu.sync_copy(dst_hbm.at[w], dstv)
    pltpu.sync_copy(zrows_hbm, deg_sh.at[pl.ds(s * RPT, RPT)])
    pltpu.sync_copy(ones_hbm, ones_v)

    plsc.subcore_barrier()

    def step(j, carry):
        pltpu.sync_copy(ones_v, deg_sh.at[dstv.at[j]], add=True)
        return carry

    lax.fori_loop(0, S, step, 0)

    plsc.subcore_barrier()

    rows = pl.ds(s * RPT, RPT)
    pltpu.sync_copy(deg_sh.at[rows], deg_hbm.at[c, rows])


def _make_sc_degree():
    mesh = plsc.VectorSubcoreMesh(core_axis_name="c", subcore_axis_name="s")
    return pl.kernel(
        _sc_degree_body,
        out_type=[jax.ShapeDtypeStruct((NC, N_PAD, D), jnp.float32)],
        mesh=mesh,
        scratch_types=[
            pltpu.VMEM((S, B), jnp.int32),
            pltpu.VMEM((B, D), jnp.float32),
            pltpu.VMEM_SHARED((N_PAD, D), jnp.float32),
        ],
    )


def _leaky(h):
    return jnp.where(h >= 0, h, 0.1 * h)


def _tc_dense_body(with_act, p_ref, d_ref, h_ref, Wl_ref, Wr_ref, b_ref,
                   LW_ref, lb_ref, o_ref):
    deg = d_ref[0, :, 0:1] + d_ref[1, :, 0:1]
    agg = (p_ref[0] + p_ref[1]) / jnp.maximum(deg, 1.0)
    t = (jnp.dot(agg, Wl_ref[...], preferred_element_type=jnp.float32)
         + b_ref[...]
         + jnp.dot(h_ref[...], Wr_ref[...], preferred_element_type=jnp.float32))
    if with_act:
        t = _leaky(t)
    t = jnp.dot(t, LW_ref[...], preferred_element_type=jnp.float32) + lb_ref[...]
    if with_act:
        t = _leaky(t)
    o_ref[...] = t


def _make_tc_dense(with_act, BN=1000):
    grid = (N // BN,)
    return pl.pallas_call(
        functools.partial(_tc_dense_body, with_act),
        grid=grid,
        in_specs=[
            pl.BlockSpec((NC, BN, D), lambda i: (0, i, 0)),
            pl.BlockSpec((NC, BN, D), lambda i: (0, i, 0)),
            pl.BlockSpec((BN, D), lambda i: (i, 0)),
            pl.BlockSpec((D, D), lambda i: (0, 0)),
            pl.BlockSpec((D, D), lambda i: (0, 0)),
            pl.BlockSpec((1, D), lambda i: (0, 0)),
            pl.BlockSpec((D, D), lambda i: (0, 0)),
            pl.BlockSpec((1, D), lambda i: (0, 0)),
        ],
        out_specs=pl.BlockSpec((BN, D), lambda i: (i, 0)),
        out_shape=jax.ShapeDtypeStruct((N, D), jnp.float32),
    )


_sc_scatter = _make_sc_scatter()
_sc_degree = _make_sc_degree()
_tc_dense_act = _make_tc_dense(True)
_tc_dense_noact = _make_tc_dense(False)


def kernel(x, edge_index,
           conv_Wl0, conv_Wr0, conv_b0, lin_W0, lin_b0,
           conv_Wl1, conv_Wr1, conv_b1, lin_W1, lin_b1,
           conv_Wl2, conv_Wr2, conv_b2, lin_W2, lin_b2):
    src = edge_index[0]
    dst = edge_index[1]
    npad = E_PAD - E
    src_pad = jnp.concatenate([src, jnp.zeros((npad,), jnp.int32)])
    dst_pad = jnp.concatenate([dst, jnp.full((npad,), N_PAD - 1, jnp.int32)])
    src_steps = src_pad.reshape(TOTAL_S, B)
    dst_steps = dst_pad.reshape(TOTAL_S, B)
    dst_slabs = dst_pad.reshape(NW, S, B)

    zrows = jnp.zeros((RPT, D), jnp.float32)
    ones = jnp.ones((B, D), jnp.float32)

    b0 = conv_b0.reshape(1, D)
    b1 = conv_b1.reshape(1, D)
    b2 = conv_b2.reshape(1, D)
    lb0 = lin_b0.reshape(1, D)
    lb1 = lin_b1.reshape(1, D)
    lb2 = lin_b2.reshape(1, D)

    (degp,) = _sc_degree(dst_slabs, zrows, ones)
    (p0,) = _sc_scatter(x, src_steps, dst_steps, zrows)
    h1 = _tc_dense_act(p0, degp, x, conv_Wl0, conv_Wr0, b0, lin_W0, lb0)
    (p1,) = _sc_scatter(h1, src_steps, dst_steps, zrows)
    h2 = _tc_dense_act(p1, degp, h1, conv_Wl1, conv_Wr1, b1, lin_W1, lb1)
    (p2,) = _sc_scatter(h2, src_steps, dst_steps, zrows)
    out = _tc_dense_noact(p2, degp, h2, conv_Wl2, conv_Wr2, b2, lin_W2, lb2)
    return out
